# baseline (speedup 1.0000x reference)
import dataclasses
import functools

import jax
import jax.numpy as jnp
from jax import lax
from jax.experimental import pallas as pl
from jax.experimental.pallas import tpu as pltpu
from jax.experimental.pallas import tpu_sc as plsc

Q = 1024
D = 16
K_REAL = 100000
NB = 50
BK = 2048
KP = NB * BK
NG1 = KP // 16
NG2 = NG1 // 16
PAD_VAL = 1e30
GROW = 384


def _stage_a_body(q_ref, kt_ref, gm_ref, gm2_ref, ksq_ref, kbf_ref, qbf_ref):
    kb = kt_ref[...]
    ksq = jnp.sum(kb * kb, axis=0, keepdims=True)
    qb = (q_ref[...] * -2.0).astype(jnp.bfloat16)
    kbb = kb.astype(jnp.bfloat16)
    kbf_ref[...] = kbb.astype(jnp.float32)
    qbf_ref[...] = q_ref[...].astype(jnp.bfloat16).astype(jnp.float32)
    dot = jnp.dot(qb, kbb, preferred_element_type=jnp.float32)
    s = dot + ksq
    m = s[:, 0:128]
    for i in range(1, 16):
        m = jnp.minimum(m, s[:, i * 128:(i + 1) * 128])
    gm_ref[...] = m
    t = jnp.minimum(m[:, 0:64], m[:, 64:128])
    t = jnp.minimum(t[:, 0:32], t[:, 32:64])
    t = jnp.minimum(t[:, 0:16], t[:, 16:32])
    t = jnp.minimum(t[:, 0:8], t[:, 8:16])
    gm2_ref[...] = t[None]
    ksq_ref[...] = ksq[None]


def _stage_a(queries, keys_t, nb):
    return pl.pallas_call(
        _stage_a_body,
        grid=(nb,),
        in_specs=[
            pl.BlockSpec((Q, D), lambda b: (0, 0)),
            pl.BlockSpec((D, BK), lambda b: (0, b)),
        ],
        out_specs=[
            pl.BlockSpec((Q, 128), lambda b: (0, b)),
            pl.BlockSpec((1, Q, 8), lambda b: (b, 0, 0)),
            pl.BlockSpec((1, 1, BK), lambda b: (b, 0, 0)),
            pl.BlockSpec((D, BK), lambda b: (0, b)),
            pl.BlockSpec((Q, D), lambda b: (0, 0)),
        ],
        out_shape=[
            jax.ShapeDtypeStruct((Q, nb * 128), jnp.float32),
            jax.ShapeDtypeStruct((nb, Q, 8), jnp.float32),
            jax.ShapeDtypeStruct((nb, 1, BK), jnp.float32),
            jax.ShapeDtypeStruct((D, nb * BK), jnp.float32),
            jax.ShapeDtypeStruct((Q, D), jnp.float32),
        ],
    )(queries, keys_t)


def _merge16(run_v, run_i, v, ids):
    sv, si = plsc.sort_key_val(v, ids, descending=True)
    take = sv < run_v
    nv = jnp.where(take, sv, run_v)
    ni = jnp.where(take, si, run_i)
    return plsc.sort_key_val(nv, ni)


NBUF = 8
NB1 = 32
NB2 = NB - NB1


def _select16(gm_v, gm2_v, nch, id_off, iota):
    run2_v = jnp.full((16,), jnp.inf, jnp.float32)
    run2_i = jnp.zeros((16,), jnp.int32)
    for c in range(nch):
        v = gm2_v[pl.ds(c * 16, 16)]
        run2_v, run2_i = _merge16(run2_v, run2_i, v, iota + (c * 16))
    child_base = ((run2_i >> 3) << 7) + (run2_i & 7)
    run1_v = jnp.full((16,), jnp.inf, jnp.float32)
    run1_i = jnp.zeros((16,), jnp.int32)
    for j in range(16):
        loc = child_base + 8 * j
        v = plsc.load_gather(gm_v, [loc])
        run1_v, run1_i = _merge16(run1_v, run1_i, v, loc + id_off)
    return run1_v, run1_i


def _sc_sel_body(gm_hbm, gm2_hbm, stv_hbm, sti_hbm, *scratch):
    gm_bufs = scratch[0:NBUF]
    gm2_bufs = scratch[NBUF:2 * NBUF]
    outd_v, outi_v = scratch[2 * NBUF:2 * NBUF + 2]
    sem_r = scratch[2 * NBUF + 2:3 * NBUF + 2]

    nc = 2
    wid = lax.axis_index("s") * nc + lax.axis_index("c")
    qbase = wid * (Q // 32)
    iota = lax.iota(jnp.int32, 16)

    def issue_rows(q, i):
        pltpu.async_copy(gm_hbm.at[q], gm_bufs[i], sem_r[i])
        pltpu.async_copy(gm2_hbm.at[q], gm2_bufs[i], sem_r[i])

    def wait_rows(q, i):
        pltpu.make_async_copy(gm_hbm.at[q], gm_bufs[i], sem_r[i]).wait()
        pltpu.make_async_copy(gm2_hbm.at[q], gm2_bufs[i], sem_r[i]).wait()

    for i in range(NBUF):
        issue_rows(qbase + i, i)

    @pl.loop(0, Q // (32 * NBUF))
    def _per_round(t):
        q0 = qbase + NBUF * t
        for i in range(NBUF):
            q = q0 + i
            wait_rows(q, i)
            r1v, r1i = _select16(gm_bufs[i], gm2_bufs[i], NB1 * 8 // 16, 0,
                                 iota)
            outd_v[...] = r1v
            outi_v[...] = r1i
            pltpu.sync_copy(outd_v, stv_hbm.at[q])
            pltpu.sync_copy(outi_v, sti_hbm.at[q])
            issue_rows(jnp.minimum(q + NBUF, Q - 1), i)

    for i in range(NBUF):
        wait_rows(qbase, i)


def _sc_fin_body(gm_hbm, gm2_hbm, stv_hbm, sti_hbm, gkeys_hbm, qc_hbm,
                 dists_hbm, idx_hbm, *scratch):
    gm_bufs = scratch[0:NBUF]
    gm2_bufs = scratch[NBUF:2 * NBUF]
    stv_bufs = scratch[2 * NBUF:3 * NBUF]
    sti_bufs = scratch[3 * NBUF:4 * NBUF]
    qc_bufs = scratch[4 * NBUF:5 * NBUF]
    tiles_bufs = scratch[5 * NBUF:6 * NBUF]
    outd_v, outi_v = scratch[6 * NBUF:6 * NBUF + 2]
    sem_r = scratch[6 * NBUF + 2:7 * NBUF + 2]
    sem_t = scratch[7 * NBUF + 2:8 * NBUF + 2]

    nc = 2
    wid = lax.axis_index("s") * nc + lax.axis_index("c")
    qbase = wid * (Q // 32)
    iota = lax.iota(jnp.int32, 16)

    def issue_rows(q, i):
        pltpu.async_copy(gm_hbm.at[q], gm_bufs[i], sem_r[i])
        pltpu.async_copy(gm2_hbm.at[q], gm2_bufs[i], sem_r[i])
        pltpu.async_copy(stv_hbm.at[q], stv_bufs[i], sem_r[i])
        pltpu.async_copy(sti_hbm.at[q], sti_bufs[i], sem_r[i])
        pltpu.async_copy(qc_hbm.at[q], qc_bufs[i], sem_r[i])

    def wait_rows(q, i):
        pltpu.make_async_copy(gm_hbm.at[q], gm_bufs[i], sem_r[i]).wait()
        pltpu.make_async_copy(gm2_hbm.at[q], gm2_bufs[i], sem_r[i]).wait()
        pltpu.make_async_copy(stv_hbm.at[q], stv_bufs[i], sem_r[i]).wait()
        pltpu.make_async_copy(sti_hbm.at[q], sti_bufs[i], sem_r[i]).wait()
        pltpu.make_async_copy(qc_hbm.at[q], qc_bufs[i], sem_r[i]).wait()

    def finish(q, tiles_v, qc_v, run1_i):
        qv = qc_v[pl.ds(0, D)]
        qbfv = qc_v[pl.ds(D, D)]
        qsq = jnp.float32(0.0)
        qbd = []
        for d in range(D):
            qd = qv[d]
            qsq = qsq + qd * qd
            qbd.append(qbfv[d])
        key_base = ((run1_i >> 7) << 11) + (run1_i & 127)

        def mm_body(tt, carry):
            runf_v, runf_i = carry
            for k in range(8):
                mm = 8 * tt + k
                ps = []
                for d in range(D):
                    row = plsc.load_gather(
                        tiles_v, [iota, jnp.full((16,), d * 16, jnp.int32) + mm])
                    ps.append(row * qbd[d])
                while len(ps) > 1:
                    ps = [ps[i] + ps[i + 1] for i in range(0, len(ps), 2)]
                ksqv = plsc.load_gather(
                    tiles_v, [iota, jnp.full((16,), D * 16, jnp.int32) + mm])
                dist = (ksqv + qsq) - 2.0 * ps[0]
                runf_v, runf_i = _merge16(runf_v, runf_i, dist,
                                          key_base + (mm << 7))
            return runf_v, runf_i

        runf_v, runf_i = lax.fori_loop(
            0, 2, mm_body,
            (jnp.full((16,), jnp.inf, jnp.float32), jnp.zeros((16,), jnp.int32)))
        outd_v[...] = runf_v
        outi_v[...] = runf_i
        pltpu.sync_copy(outd_v, dists_hbm.at[q])
        pltpu.sync_copy(outi_v, idx_hbm.at[q])

    for i in range(NBUF):
        issue_rows(qbase + i, i)

    @pl.loop(0, Q // (32 * NBUF))
    def _per_round(t):
        q0 = qbase + NBUF * t
        r1s = []
        hs = []
        for i in range(NBUF):
            q = q0 + i
            wait_rows(q, i)
            r1v, r1i = _select16(gm_bufs[i], gm2_bufs[i], NB2 * 8 // 16,
                                 NB1 * 128, iota)
            r1v, r1i = _merge16(r1v, r1i, stv_bufs[i][...], sti_bufs[i][...])
            hs.append(pltpu.async_copy(gkeys_hbm.at[r1i], tiles_bufs[i],
                                       sem_t[i]))
            r1s.append(r1i)
        for i in range(NBUF):
            q = q0 + i
            hs[i].wait()
            finish(q, tiles_bufs[i], qc_bufs[i], r1s[i])
            issue_rows(jnp.minimum(q + NBUF, Q - 1), i)

    for i in range(NBUF):
        wait_rows(qbase, i)


def _sc_compiler_params():
    cp = pltpu.CompilerParams()
    if "needs_layout_passes" in pltpu.CompilerParams.__dataclass_fields__:
        cp = dataclasses.replace(cp, needs_layout_passes=False)
    return cp


def _stage_b1(gm1, gm21):
    mesh = plsc.VectorSubcoreMesh(core_axis_name="c", subcore_axis_name="s")
    f = pl.kernel(
        _sc_sel_body,
        out_type=(jax.ShapeDtypeStruct((Q, 16), jnp.float32),
                  jax.ShapeDtypeStruct((Q, 16), jnp.int32)),
        mesh=mesh,
        scratch_types=(
            [pltpu.VMEM((NB1 * 128,), jnp.float32)] * NBUF
            + [pltpu.VMEM((NB1 * 8,), jnp.float32)] * NBUF
            + [pltpu.VMEM((16,), jnp.float32), pltpu.VMEM((16,), jnp.int32)]
            + [pltpu.SemaphoreType.DMA] * NBUF
        ),
        compiler_params=_sc_compiler_params(),
    )
    return f(gm1, gm21)


def _stage_b2(gm2_, gm22, stv, sti, gkeys, qc):
    mesh = plsc.VectorSubcoreMesh(core_axis_name="c", subcore_axis_name="s")
    f = pl.kernel(
        _sc_fin_body,
        out_type=(jax.ShapeDtypeStruct((Q, 16), jnp.float32),
                  jax.ShapeDtypeStruct((Q, 16), jnp.int32)),
        mesh=mesh,
        scratch_types=(
            [pltpu.VMEM((NB2 * 128,), jnp.float32)] * NBUF
            + [pltpu.VMEM((NB2 * 8,), jnp.float32)] * NBUF
            + [pltpu.VMEM((16,), jnp.float32)] * NBUF
            + [pltpu.VMEM((16,), jnp.int32)] * NBUF
            + [pltpu.VMEM((2 * D,), jnp.float32)] * NBUF
            + [pltpu.VMEM((16, GROW), jnp.float32)] * NBUF
            + [pltpu.VMEM((16,), jnp.float32), pltpu.VMEM((16,), jnp.int32)]
            + [pltpu.SemaphoreType.DMA] * (2 * NBUF)
        ),
        compiler_params=_sc_compiler_params(),
    )
    return f(gm2_, gm22, stv, sti, gkeys, qc)


def _assemble_half(gm2_3d, ksq_3d, kbf_t, nb):
    ng1 = nb * 128
    gm2 = jnp.transpose(gm2_3d, (1, 0, 2)).reshape(Q, nb * 8)
    ksq_g = (ksq_3d.reshape(nb, 16, 128).transpose(0, 2, 1)
             .reshape(ng1, 1, 16))
    kbf = (kbf_t.T.reshape(nb, 16, 128, D).transpose(0, 2, 3, 1)
           .reshape(ng1, D, 16))
    gkeys = jnp.concatenate([kbf, ksq_g], axis=1).reshape(ng1, (D + 1) * 16)
    return gm2, jnp.pad(gkeys, ((0, 0), (0, GROW - (D + 1) * 16)))


def kernel(queries, keys, k):
    del k
    keys_pad = jnp.concatenate(
        [keys, jnp.full((KP - K_REAL, D), PAD_VAL, jnp.float32)], axis=0)
    keys_t = keys_pad.T

    gm1, gm2_3d1, ksq_1, kbf_t1, queries_bf = _stage_a(
        queries, keys_t[:, :NB1 * BK], NB1)
    gm21, gkeys1 = _assemble_half(gm2_3d1, ksq_1, kbf_t1, NB1)
    stv, sti = _stage_b1(gm1, gm21)

    gm_2, gm2_3d2, ksq_2, kbf_t2, _ = _stage_a(
        queries, keys_t[:, NB1 * BK:], NB2)
    gm22, gkeys2 = _assemble_half(gm2_3d2, ksq_2, kbf_t2, NB2)
    gkeys = jnp.concatenate([gkeys1, gkeys2], axis=0)

    qc = jnp.concatenate([queries, queries_bf], axis=1)
    dists, idx = _stage_b2(gm_2, gm22, stv, sti, gkeys, qc)
    return (dists, idx)


# speedup vs baseline: 1.0015x; 1.0015x over previous
import dataclasses
import functools

import jax
import jax.numpy as jnp
from jax import lax
from jax.experimental import pallas as pl
from jax.experimental.pallas import tpu as pltpu
from jax.experimental.pallas import tpu_sc as plsc

Q = 1024
D = 16
K_REAL = 100000
NB = 50
BK = 2048
KP = NB * BK
NG1 = KP // 16
NG2 = NG1 // 16
PAD_VAL = 1e30
GROW = 384


def _stage_a_body(q_ref, kt_ref, gm_ref, gm2_ref, ksq_ref, kbf_ref, qbf_ref):
    kb = kt_ref[...]
    ksq = jnp.sum(kb * kb, axis=0, keepdims=True)
    qb = (q_ref[...] * -2.0).astype(jnp.bfloat16)
    kbb = kb.astype(jnp.bfloat16)
    kbf_ref[...] = kbb.astype(jnp.float32)
    qbf_ref[...] = q_ref[...].astype(jnp.bfloat16).astype(jnp.float32)
    dot = jnp.dot(qb, kbb, preferred_element_type=jnp.float32)
    s = dot + ksq
    m = s[:, 0:128]
    for i in range(1, 16):
        m = jnp.minimum(m, s[:, i * 128:(i + 1) * 128])
    gm_ref[...] = m
    t = jnp.minimum(m[:, 0:64], m[:, 64:128])
    t = jnp.minimum(t[:, 0:32], t[:, 32:64])
    t = jnp.minimum(t[:, 0:16], t[:, 16:32])
    t = jnp.minimum(t[:, 0:8], t[:, 8:16])
    gm2_ref[...] = t[None]
    ksq_ref[...] = ksq[None]


def _stage_a(queries, keys_t, nb):
    return pl.pallas_call(
        _stage_a_body,
        grid=(nb,),
        in_specs=[
            pl.BlockSpec((Q, D), lambda b: (0, 0)),
            pl.BlockSpec((D, BK), lambda b: (0, b)),
        ],
        out_specs=[
            pl.BlockSpec((Q, 128), lambda b: (0, b)),
            pl.BlockSpec((1, Q, 8), lambda b: (b, 0, 0)),
            pl.BlockSpec((1, 1, BK), lambda b: (b, 0, 0)),
            pl.BlockSpec((D, BK), lambda b: (0, b)),
            pl.BlockSpec((Q, D), lambda b: (0, 0)),
        ],
        out_shape=[
            jax.ShapeDtypeStruct((Q, nb * 128), jnp.float32),
            jax.ShapeDtypeStruct((nb, Q, 8), jnp.float32),
            jax.ShapeDtypeStruct((nb, 1, BK), jnp.float32),
            jax.ShapeDtypeStruct((D, nb * BK), jnp.float32),
            jax.ShapeDtypeStruct((Q, D), jnp.float32),
        ],
    )(queries, keys_t)


def _merge16(run_v, run_i, v, ids):
    sv, si = plsc.sort_key_val(v, ids, descending=True)
    take = sv < run_v
    nv = jnp.where(take, sv, run_v)
    ni = jnp.where(take, si, run_i)
    return plsc.sort_key_val(nv, ni)


NBUF = 4
NB1 = 32
NB2 = NB - NB1


def _select16(gm_v, gm2_v, nch, id_off, iota):
    run2_v = jnp.full((16,), jnp.inf, jnp.float32)
    run2_i = jnp.zeros((16,), jnp.int32)
    for c in range(nch):
        v = gm2_v[pl.ds(c * 16, 16)]
        run2_v, run2_i = _merge16(run2_v, run2_i, v, iota + (c * 16))
    child_base = ((run2_i >> 3) << 7) + (run2_i & 7)
    run1_v = jnp.full((16,), jnp.inf, jnp.float32)
    run1_i = jnp.zeros((16,), jnp.int32)
    for j in range(16):
        loc = child_base + 8 * j
        v = plsc.load_gather(gm_v, [loc])
        run1_v, run1_i = _merge16(run1_v, run1_i, v, loc + id_off)
    return run1_v, run1_i


def _sc_sel_body(gm_hbm, gm2_hbm, stv_hbm, sti_hbm, *scratch):
    gm_bufs = scratch[0:NBUF]
    gm2_bufs = scratch[NBUF:2 * NBUF]
    outd_v, outi_v = scratch[2 * NBUF:2 * NBUF + 2]
    sem_r = scratch[2 * NBUF + 2:3 * NBUF + 2]

    nc = 2
    wid = lax.axis_index("s") * nc + lax.axis_index("c")
    qbase = wid * (Q // 32)
    iota = lax.iota(jnp.int32, 16)

    def issue_rows(q, i):
        pltpu.async_copy(gm_hbm.at[q], gm_bufs[i], sem_r[i])
        pltpu.async_copy(gm2_hbm.at[q], gm2_bufs[i], sem_r[i])

    def wait_rows(q, i):
        pltpu.make_async_copy(gm_hbm.at[q], gm_bufs[i], sem_r[i]).wait()
        pltpu.make_async_copy(gm2_hbm.at[q], gm2_bufs[i], sem_r[i]).wait()

    for i in range(NBUF):
        issue_rows(qbase + i, i)

    @pl.loop(0, Q // (32 * NBUF))
    def _per_round(t):
        q0 = qbase + NBUF * t
        for i in range(NBUF):
            q = q0 + i
            wait_rows(q, i)
            r1v, r1i = _select16(gm_bufs[i], gm2_bufs[i], NB1 * 8 // 16, 0,
                                 iota)
            outd_v[...] = r1v
            outi_v[...] = r1i
            pltpu.sync_copy(outd_v, stv_hbm.at[q])
            pltpu.sync_copy(outi_v, sti_hbm.at[q])
            issue_rows(jnp.minimum(q + NBUF, Q - 1), i)

    for i in range(NBUF):
        wait_rows(qbase, i)


def _sc_fin_body(gm_hbm, gm2_hbm, stv_hbm, sti_hbm, gkeys_hbm, qc_hbm,
                 dists_hbm, idx_hbm, *scratch):
    gm_bufs = scratch[0:NBUF]
    gm2_bufs = scratch[NBUF:2 * NBUF]
    stv_bufs = scratch[2 * NBUF:3 * NBUF]
    sti_bufs = scratch[3 * NBUF:4 * NBUF]
    qc_bufs = scratch[4 * NBUF:5 * NBUF]
    tiles_bufs = scratch[5 * NBUF:6 * NBUF]
    outd_v, outi_v = scratch[6 * NBUF:6 * NBUF + 2]
    sem_r = scratch[6 * NBUF + 2:7 * NBUF + 2]
    sem_t = scratch[7 * NBUF + 2:8 * NBUF + 2]

    nc = 2
    wid = lax.axis_index("s") * nc + lax.axis_index("c")
    qbase = wid * (Q // 32)
    iota = lax.iota(jnp.int32, 16)

    def issue_rows(q, i):
        pltpu.async_copy(gm_hbm.at[q], gm_bufs[i], sem_r[i])
        pltpu.async_copy(gm2_hbm.at[q], gm2_bufs[i], sem_r[i])
        pltpu.async_copy(stv_hbm.at[q], stv_bufs[i], sem_r[i])
        pltpu.async_copy(sti_hbm.at[q], sti_bufs[i], sem_r[i])
        pltpu.async_copy(qc_hbm.at[q], qc_bufs[i], sem_r[i])

    def wait_rows(q, i):
        pltpu.make_async_copy(gm_hbm.at[q], gm_bufs[i], sem_r[i]).wait()
        pltpu.make_async_copy(gm2_hbm.at[q], gm2_bufs[i], sem_r[i]).wait()
        pltpu.make_async_copy(stv_hbm.at[q], stv_bufs[i], sem_r[i]).wait()
        pltpu.make_async_copy(sti_hbm.at[q], sti_bufs[i], sem_r[i]).wait()
        pltpu.make_async_copy(qc_hbm.at[q], qc_bufs[i], sem_r[i]).wait()

    def finish(q, tiles_v, qc_v, run1_i):
        qv = qc_v[pl.ds(0, D)]
        qbfv = qc_v[pl.ds(D, D)]
        qsq = jnp.float32(0.0)
        qbd = []
        for d in range(D):
            qd = qv[d]
            qsq = qsq + qd * qd
            qbd.append(qbfv[d])
        key_base = ((run1_i >> 7) << 11) + (run1_i & 127)

        def mm_body(tt, carry):
            runf_v, runf_i = carry
            for k in range(8):
                mm = 8 * tt + k
                ps = []
                for d in range(D):
                    row = plsc.load_gather(
                        tiles_v, [iota, jnp.full((16,), d * 16, jnp.int32) + mm])
                    ps.append(row * qbd[d])
                while len(ps) > 1:
                    ps = [ps[i] + ps[i + 1] for i in range(0, len(ps), 2)]
                ksqv = plsc.load_gather(
                    tiles_v, [iota, jnp.full((16,), D * 16, jnp.int32) + mm])
                dist = (ksqv + qsq) - 2.0 * ps[0]
                runf_v, runf_i = _merge16(runf_v, runf_i, dist,
                                          key_base + (mm << 7))
            return runf_v, runf_i

        runf_v, runf_i = lax.fori_loop(
            0, 2, mm_body,
            (jnp.full((16,), jnp.inf, jnp.float32), jnp.zeros((16,), jnp.int32)))
        outd_v[...] = runf_v
        outi_v[...] = runf_i
        pltpu.sync_copy(outd_v, dists_hbm.at[q])
        pltpu.sync_copy(outi_v, idx_hbm.at[q])

    for i in range(NBUF):
        issue_rows(qbase + i, i)

    @pl.loop(0, Q // (32 * NBUF))
    def _per_round(t):
        q0 = qbase + NBUF * t
        r1s = []
        hs = []
        for i in range(NBUF):
            q = q0 + i
            wait_rows(q, i)
            r1v, r1i = _select16(gm_bufs[i], gm2_bufs[i], NB2 * 8 // 16,
                                 NB1 * 128, iota)
            r1v, r1i = _merge16(r1v, r1i, stv_bufs[i][...], sti_bufs[i][...])
            hs.append(pltpu.async_copy(gkeys_hbm.at[r1i], tiles_bufs[i],
                                       sem_t[i]))
            r1s.append(r1i)
        for i in range(NBUF):
            q = q0 + i
            hs[i].wait()
            finish(q, tiles_bufs[i], qc_bufs[i], r1s[i])
            issue_rows(jnp.minimum(q + NBUF, Q - 1), i)

    for i in range(NBUF):
        wait_rows(qbase, i)


def _sc_compiler_params():
    cp = pltpu.CompilerParams()
    if "needs_layout_passes" in pltpu.CompilerParams.__dataclass_fields__:
        cp = dataclasses.replace(cp, needs_layout_passes=False)
    return cp


def _stage_b1(gm1, gm21):
    mesh = plsc.VectorSubcoreMesh(core_axis_name="c", subcore_axis_name="s")
    f = pl.kernel(
        _sc_sel_body,
        out_type=(jax.ShapeDtypeStruct((Q, 16), jnp.float32),
                  jax.ShapeDtypeStruct((Q, 16), jnp.int32)),
        mesh=mesh,
        scratch_types=(
            [pltpu.VMEM((NB1 * 128,), jnp.float32)] * NBUF
            + [pltpu.VMEM((NB1 * 8,), jnp.float32)] * NBUF
            + [pltpu.VMEM((16,), jnp.float32), pltpu.VMEM((16,), jnp.int32)]
            + [pltpu.SemaphoreType.DMA] * NBUF
        ),
        compiler_params=_sc_compiler_params(),
    )
    return f(gm1, gm21)


def _stage_b2(gm2_, gm22, stv, sti, gkeys, qc):
    mesh = plsc.VectorSubcoreMesh(core_axis_name="c", subcore_axis_name="s")
    f = pl.kernel(
        _sc_fin_body,
        out_type=(jax.ShapeDtypeStruct((Q, 16), jnp.float32),
                  jax.ShapeDtypeStruct((Q, 16), jnp.int32)),
        mesh=mesh,
        scratch_types=(
            [pltpu.VMEM((NB2 * 128,), jnp.float32)] * NBUF
            + [pltpu.VMEM((NB2 * 8,), jnp.float32)] * NBUF
            + [pltpu.VMEM((16,), jnp.float32)] * NBUF
            + [pltpu.VMEM((16,), jnp.int32)] * NBUF
            + [pltpu.VMEM((2 * D,), jnp.float32)] * NBUF
            + [pltpu.VMEM((16, GROW), jnp.float32)] * NBUF
            + [pltpu.VMEM((16,), jnp.float32), pltpu.VMEM((16,), jnp.int32)]
            + [pltpu.SemaphoreType.DMA] * (2 * NBUF)
        ),
        compiler_params=_sc_compiler_params(),
    )
    return f(gm2_, gm22, stv, sti, gkeys, qc)


def _assemble_half(gm2_3d, ksq_3d, kbf_t, nb):
    ng1 = nb * 128
    gm2 = jnp.transpose(gm2_3d, (1, 0, 2)).reshape(Q, nb * 8)
    ksq_g = (ksq_3d.reshape(nb, 16, 128).transpose(0, 2, 1)
             .reshape(ng1, 1, 16))
    kbf = (kbf_t.T.reshape(nb, 16, 128, D).transpose(0, 2, 3, 1)
           .reshape(ng1, D, 16))
    gkeys = jnp.concatenate([kbf, ksq_g], axis=1).reshape(ng1, (D + 1) * 16)
    return gm2, jnp.pad(gkeys, ((0, 0), (0, GROW - (D + 1) * 16)))


def kernel(queries, keys, k):
    del k
    keys_pad = jnp.concatenate(
        [keys, jnp.full((KP - K_REAL, D), PAD_VAL, jnp.float32)], axis=0)
    keys_t = keys_pad.T

    gm1, gm2_3d1, ksq_1, kbf_t1, queries_bf = _stage_a(
        queries, keys_t[:, :NB1 * BK], NB1)
    gm21, gkeys1 = _assemble_half(gm2_3d1, ksq_1, kbf_t1, NB1)
    stv, sti = _stage_b1(gm1, gm21)

    gm_2, gm2_3d2, ksq_2, kbf_t2, _ = _stage_a(
        queries, keys_t[:, NB1 * BK:], NB2)
    gm22, gkeys2 = _assemble_half(gm2_3d2, ksq_2, kbf_t2, NB2)
    gkeys = jnp.concatenate([gkeys1, gkeys2], axis=0)

    qc = jnp.concatenate([queries, queries_bf], axis=1)
    dists, idx = _stage_b2(gm_2, gm22, stv, sti, gkeys, qc)
    return (dists, idx)
